# baseline (speedup 1.0000x reference)
import functools

import jax
import jax.numpy as jnp
from jax import lax
from jax.experimental import pallas as pl
from jax.experimental.pallas import tpu as pltpu
from jax.experimental.pallas import tpu_sc as plsc

MAXD = 10
V = MAXD + 1
D = 64
N_SIDE = 1024
H = N_SIDE // 2
B = N_SIDE * N_SIDE
B2 = B // 2
NC = 2
NS = 16
NW = NC * NS
K = 128
NKC = B2 // (NW * K)
NROW = N_SIDE // NW
L = 16
NBUF = 4


def _sc_body(dist_hbm, table2_hbm, out_hbm, idx_v, cidx_v, table2_sh, *bufs_and_sems):
    rows = bufs_and_sems[:NBUF]
    sg = bufs_and_sems[NBUF : 2 * NBUF]
    so = bufs_and_sems[2 * NBUF : 3 * NBUF]

    sid = lax.axis_index("s")
    wid = sid * NC + lax.axis_index("c")
    row0 = wid * NKC

    @pl.when(sid == 0)
    def _():
        pltpu.sync_copy(table2_hbm, table2_sh)

    pltpu.sync_copy(dist_hbm.at[pl.ds(wid * NROW, NROW)], idx_v)

    def combine_row(il, carry):
        for g in range(H // L):
            a = idx_v[il, pl.ds(g * L, L)]
            b = idx_v[il, pl.ds(H + g * L, L)]
            a = jnp.minimum(jnp.maximum(a, 0), MAXD)
            b = jnp.minimum(jnp.maximum(b, 0), MAXD)
            cidx_v[il * (H // K) + g // (K // L), pl.ds((g % (K // L)) * L, L)] = (
                a * V + b
            )
        return carry

    lax.fori_loop(0, NROW, combine_row, 0)

    plsc.subcore_barrier()

    def fire_gather(j, b):
        pltpu.async_copy(table2_sh.at[cidx_v.at[j]], rows[b], sg[b])

    def fire_write(j, b):
        pltpu.async_copy(rows[b], out_hbm.at[pl.ds((row0 + j) * K, K)], so[b])

    def wait_gather(j, b):
        pltpu.make_async_copy(table2_sh.at[cidx_v.at[j]], rows[b], sg[b]).wait()

    def wait_write(j, b):
        pltpu.make_async_copy(
            rows[b], out_hbm.at[pl.ds((row0 + j) * K, K)], so[b]
        ).wait()

    for b in range(NBUF):
        fire_gather(b, b)

    def outer(gi, carry):
        g = gi * NBUF
        for b in range(NBUF):
            j = g + b
            wait_gather(j, b)
            fire_write(j, b)
            wait_write(j, b)

            @pl.when(j + NBUF < NKC)
            def _():
                fire_gather(j + NBUF, b)

        return carry

    lax.fori_loop(0, NKC // NBUF, outer, 0)


_gather_call = functools.partial(
    pl.kernel,
    out_type=jax.ShapeDtypeStruct((B2, 2 * D), jnp.float32),
    mesh=plsc.VectorSubcoreMesh(
        core_axis_name="c", subcore_axis_name="s", num_cores=NC, num_subcores=NS
    ),
    scratch_types=(
        [
            pltpu.VMEM((NROW, N_SIDE), jnp.int32),
            pltpu.VMEM((NKC, K), jnp.int32),
            pltpu.VMEM_SHARED((V * V, 2 * D), jnp.float32),
        ]
        + [pltpu.VMEM((K, 2 * D), jnp.float32)] * NBUF
        + [pltpu.SemaphoreType.DMA] * (2 * NBUF)
    ),
)(_sc_body)


def _relayout_body(x_ref, o_ref):
    eye = jnp.eye(128, dtype=jnp.float32)
    for c in range(H // 128):
        t = lax.dot_general(
            x_ref[pl.ds(c * 128, 128), :],
            eye,
            (((0,), (0,)), ((), ())),
            preferred_element_type=jnp.float32,
        )
        o_ref[0, :, pl.ds(c * 128, 128)] = t[:D, :]
        o_ref[0, :, pl.ds(H + c * 128, 128)] = t[D:, :]


_relayout_call = pl.pallas_call(
    _relayout_body,
    grid=(N_SIDE,),
    in_specs=[pl.BlockSpec((H, 2 * D), lambda i: (i, 0))],
    out_specs=pl.BlockSpec((1, D, N_SIDE), lambda i: (i, 0, 0)),
    out_shape=jax.ShapeDtypeStruct((N_SIDE, D, N_SIDE), jnp.float32),
)


def kernel(distance_matrix, table):
    table2 = jnp.concatenate(
        [jnp.repeat(table, V, axis=0), jnp.tile(table, (V, 1))], axis=1
    )
    paired = _gather_call(distance_matrix, table2)
    out_t = _relayout_call(paired)
    return jnp.transpose(out_t, (0, 2, 1))


# speedup vs baseline: 1.1631x; 1.1631x over previous
import functools

import jax
import jax.numpy as jnp
from jax import lax
from jax.experimental import pallas as pl
from jax.experimental.pallas import tpu as pltpu
from jax.experimental.pallas import tpu_sc as plsc

MAXD = 10
V = MAXD + 1
D = 64
N_SIDE = 1024
H = N_SIDE // 2
TOP = N_SIDE // 2
NC = 2
NS = 16
NW = NC * NS
K = 128
NROW = TOP // NW
NKC = TOP * H // (NW * K)
L = 16
NBUF = 4
VP = 16


def _sc_body(dist_hbm, table2_hbm, out_hbm, idx_v, cidx_v, table2_sh, *bufs_and_sems):
    rows = bufs_and_sems[:NBUF]
    sg = bufs_and_sems[NBUF : 2 * NBUF]
    so = bufs_and_sems[2 * NBUF : 3 * NBUF]

    sid = lax.axis_index("s")
    wid = sid * NC + lax.axis_index("c")
    row0 = wid * NKC

    @pl.when(sid == 0)
    def _():
        pltpu.sync_copy(table2_hbm, table2_sh)

    pltpu.sync_copy(dist_hbm.at[pl.ds(wid * NROW, NROW)], idx_v)

    def combine_row(il, carry):
        for g in range(H // L):
            a = idx_v[il, pl.ds(g * L, L)]
            b = idx_v[il, pl.ds(H + g * L, L)]
            a = jnp.minimum(jnp.maximum(a, 0), MAXD)
            b = jnp.minimum(jnp.maximum(b, 0), MAXD)
            cidx_v[il * (H // K) + g // (K // L), pl.ds((g % (K // L)) * L, L)] = (
                a * V + b
            )
        return carry

    lax.fori_loop(0, NROW, combine_row, 0)

    plsc.subcore_barrier()

    def fire_gather(j, b):
        pltpu.async_copy(table2_sh.at[cidx_v.at[j]], rows[b], sg[b])

    def fire_write(j, b):
        pltpu.async_copy(rows[b], out_hbm.at[pl.ds((row0 + j) * K, K)], so[b])

    def wait_gather(j, b):
        pltpu.make_async_copy(table2_sh.at[cidx_v.at[j]], rows[b], sg[b]).wait()

    def wait_write(j, b):
        pltpu.make_async_copy(
            rows[b], out_hbm.at[pl.ds((row0 + j) * K, K)], so[b]
        ).wait()

    for b in range(NBUF):
        fire_gather(b, b)

    def outer(gi, carry):
        g = gi * NBUF
        for b in range(NBUF):
            j = g + b
            wait_gather(j, b)
            fire_write(j, b)
            wait_write(j, b)

            @pl.when(j + NBUF < NKC)
            def _():
                fire_gather(j + NBUF, b)

        return carry

    lax.fori_loop(0, NKC // NBUF, outer, 0)


_gather_call = functools.partial(
    pl.kernel,
    out_type=jax.ShapeDtypeStruct((TOP * H, 2 * D), jnp.float32),
    mesh=plsc.VectorSubcoreMesh(
        core_axis_name="c", subcore_axis_name="s", num_cores=NC, num_subcores=NS
    ),
    scratch_types=(
        [
            pltpu.VMEM((NROW, N_SIDE), jnp.int32),
            pltpu.VMEM((NKC, K), jnp.int32),
            pltpu.VMEM_SHARED((V * V, 2 * D), jnp.float32),
        ]
        + [pltpu.VMEM((K, 2 * D), jnp.float32)] * NBUF
        + [pltpu.SemaphoreType.DMA] * (2 * NBUF)
    ),
)(_sc_body)


def _tc_body(x_ref, d_ref, tt_ref, o_ref):
    i = pl.program_id(0)

    @pl.when(i < TOP)
    def _():
        for c in range(H // 128):
            t = x_ref[pl.ds(c * 128, 128), :].T
            o_ref[0, :, pl.ds(c * 128, 128)] = t[:D, :]
            o_ref[0, :, pl.ds(H + c * 128, 128)] = t[D:, :]

    @pl.when(i >= TOP)
    def _():
        idx = d_ref[0, 0, :]
        idx = jnp.minimum(jnp.maximum(idx, 0), MAXD)
        onehot = (
            lax.broadcasted_iota(jnp.int32, (VP, N_SIDE), 0) == idx[None, :]
        ).astype(jnp.float32)
        o_ref[0] = lax.dot_general(
            tt_ref[...],
            onehot,
            (((1,), (0,)), ((), ())),
            precision=lax.Precision.HIGHEST,
            preferred_element_type=jnp.float32,
        )


_relayout_call = pl.pallas_call(
    _tc_body,
    grid=(N_SIDE,),
    in_specs=[
        pl.BlockSpec((H, 2 * D), lambda i: (jnp.minimum(i, TOP - 1), 0)),
        pl.BlockSpec((1, 1, N_SIDE), lambda i: (i, 0, 0)),
        pl.BlockSpec((D, VP), lambda i: (0, 0)),
    ],
    out_specs=pl.BlockSpec((1, D, N_SIDE), lambda i: (i, 0, 0)),
    out_shape=jax.ShapeDtypeStruct((N_SIDE, D, N_SIDE), jnp.float32),
)


def kernel(distance_matrix, table):
    table2 = jnp.concatenate(
        [jnp.repeat(table, V, axis=0), jnp.tile(table, (V, 1))], axis=1
    )
    paired = _gather_call(distance_matrix, table2)
    tableT = jnp.pad(table, ((0, VP - V), (0, 0))).T
    dist3 = distance_matrix.reshape(N_SIDE, 1, N_SIDE)
    out_t = _relayout_call(paired, dist3, tableT)
    return jnp.transpose(out_t, (0, 2, 1))


# speedup vs baseline: 1.2268x; 1.0548x over previous
import functools

import jax
import jax.numpy as jnp
from jax import lax
from jax.experimental import pallas as pl
from jax.experimental.pallas import tpu as pltpu
from jax.experimental.pallas import tpu_sc as plsc

MAXD = 10
V = MAXD + 1
D = 64
N_SIDE = 1024
H = N_SIDE // 2
TOP = N_SIDE // 4
NC = 2
NS = 16
NW = NC * NS
K = 128
NROW = TOP // NW
NKC = TOP * H // (NW * K)
L = 16
NBUF = 4
VP = 16


def _sc_body(dist_hbm, table2_hbm, out_hbm, idx_v, cidx_v, table2_sh, *bufs_and_sems):
    rows = bufs_and_sems[:NBUF]
    sg = bufs_and_sems[NBUF : 2 * NBUF]
    so = bufs_and_sems[2 * NBUF : 3 * NBUF]

    sid = lax.axis_index("s")
    wid = sid * NC + lax.axis_index("c")
    row0 = wid * NKC

    @pl.when(sid == 0)
    def _():
        pltpu.sync_copy(table2_hbm, table2_sh)

    pltpu.sync_copy(dist_hbm.at[pl.ds(wid * NROW, NROW)], idx_v)

    def combine_row(il, carry):
        for g in range(H // L):
            a = idx_v[il, pl.ds(g * L, L)]
            b = idx_v[il, pl.ds(H + g * L, L)]
            a = jnp.minimum(jnp.maximum(a, 0), MAXD)
            b = jnp.minimum(jnp.maximum(b, 0), MAXD)
            cidx_v[il * (H // K) + g // (K // L), pl.ds((g % (K // L)) * L, L)] = (
                a * V + b
            )
        return carry

    lax.fori_loop(0, NROW, combine_row, 0)

    plsc.subcore_barrier()

    def fire_gather(j, b):
        pltpu.async_copy(table2_sh.at[cidx_v.at[j]], rows[b], sg[b])

    def fire_write(j, b):
        pltpu.async_copy(rows[b], out_hbm.at[pl.ds((row0 + j) * K, K)], so[b])

    def wait_gather(j, b):
        pltpu.make_async_copy(table2_sh.at[cidx_v.at[j]], rows[b], sg[b]).wait()

    def wait_write(j, b):
        pltpu.make_async_copy(
            rows[b], out_hbm.at[pl.ds((row0 + j) * K, K)], so[b]
        ).wait()

    for b in range(NBUF):
        fire_gather(b, b)

    def outer(gi, carry):
        g = gi * NBUF
        for b in range(NBUF):
            j = g + b
            wait_gather(j, b)
            fire_write(j, b)
            wait_write(j, b)

            @pl.when(j + NBUF < NKC)
            def _():
                fire_gather(j + NBUF, b)

        return carry

    lax.fori_loop(0, NKC // NBUF, outer, 0)


_gather_call = functools.partial(
    pl.kernel,
    out_type=jax.ShapeDtypeStruct((TOP * H, 2 * D), jnp.float32),
    mesh=plsc.VectorSubcoreMesh(
        core_axis_name="c", subcore_axis_name="s", num_cores=NC, num_subcores=NS
    ),
    scratch_types=(
        [
            pltpu.VMEM((NROW, N_SIDE), jnp.int32),
            pltpu.VMEM((NKC, K), jnp.int32),
            pltpu.VMEM_SHARED((V * V, 2 * D), jnp.float32),
        ]
        + [pltpu.VMEM((K, 2 * D), jnp.float32)] * NBUF
        + [pltpu.SemaphoreType.DMA] * (2 * NBUF)
    ),
)(_sc_body)


def _tc_body(x_ref, d_ref, tt_ref, o_ref):
    i = pl.program_id(0)

    @pl.when(i < TOP)
    def _():
        for c in range(H // 128):
            t = x_ref[pl.ds(c * 128, 128), :].T
            o_ref[0, :, pl.ds(c * 128, 128)] = t[:D, :]
            o_ref[0, :, pl.ds(H + c * 128, 128)] = t[D:, :]

    @pl.when(i >= TOP)
    def _():
        idx = d_ref[0, 0, :]
        idx = jnp.minimum(jnp.maximum(idx, 0), MAXD)
        onehot = (
            lax.broadcasted_iota(jnp.int32, (VP, N_SIDE), 0) == idx[None, :]
        ).astype(jnp.float32)
        o_ref[0] = lax.dot_general(
            tt_ref[...],
            onehot,
            (((1,), (0,)), ((), ())),
            precision=lax.Precision.HIGHEST,
            preferred_element_type=jnp.float32,
        )


_relayout_call = pl.pallas_call(
    _tc_body,
    grid=(N_SIDE,),
    in_specs=[
        pl.BlockSpec((H, 2 * D), lambda i: (jnp.minimum(i, TOP - 1), 0)),
        pl.BlockSpec((1, 1, N_SIDE), lambda i: (i, 0, 0)),
        pl.BlockSpec((D, VP), lambda i: (0, 0)),
    ],
    out_specs=pl.BlockSpec((1, D, N_SIDE), lambda i: (i, 0, 0)),
    out_shape=jax.ShapeDtypeStruct((N_SIDE, D, N_SIDE), jnp.float32),
)


def kernel(distance_matrix, table):
    table2 = jnp.concatenate(
        [jnp.repeat(table, V, axis=0), jnp.tile(table, (V, 1))], axis=1
    )
    paired = _gather_call(distance_matrix, table2)
    tableT = jnp.pad(table, ((0, VP - V), (0, 0))).T
    dist3 = distance_matrix.reshape(N_SIDE, 1, N_SIDE)
    out_t = _relayout_call(paired, dist3, tableT)
    return jnp.transpose(out_t, (0, 2, 1))


# speedup vs baseline: 2.0722x; 1.6892x over previous
import functools

import jax
import jax.numpy as jnp
from jax import lax
from jax.experimental import pallas as pl
from jax.experimental.pallas import tpu as pltpu
from jax.experimental.pallas import tpu_sc as plsc

MAXD = 10
V = MAXD + 1
D = 64
N_SIDE = 1024
H = N_SIDE // 2
TOP = N_SIDE // 4
NC = 2
NS = 16
NW = NC * NS
K = 128
NROW = TOP // NW
NKC = TOP * H // (NW * K)
L = 16
NBUF = 4
VP = 16


def _sc_body(dist_hbm, table2_hbm, out_hbm, idx_v, cidx_v, table2_sh, *bufs_and_sems):
    rows = bufs_and_sems[:NBUF]
    sg = bufs_and_sems[NBUF : 2 * NBUF]
    so = bufs_and_sems[2 * NBUF : 3 * NBUF]

    sid = lax.axis_index("s")
    wid = sid * NC + lax.axis_index("c")
    row0 = wid * NKC

    @pl.when(sid == 0)
    def _():
        pltpu.sync_copy(table2_hbm, table2_sh)

    pltpu.sync_copy(dist_hbm.at[pl.ds(wid * NROW, NROW)], idx_v)

    def combine_row(il, carry):
        for g in range(H // L):
            a = idx_v[il, pl.ds(g * L, L)]
            b = idx_v[il, pl.ds(H + g * L, L)]
            a = jnp.minimum(jnp.maximum(a, 0), MAXD)
            b = jnp.minimum(jnp.maximum(b, 0), MAXD)
            cidx_v[il * (H // K) + g // (K // L), pl.ds((g % (K // L)) * L, L)] = (
                a * V + b
            )
        return carry

    lax.fori_loop(0, NROW, combine_row, 0)

    plsc.subcore_barrier()

    def fire_gather(j, b):
        pltpu.async_copy(table2_sh.at[cidx_v.at[j]], rows[b], sg[b])

    def fire_write(j, b):
        pltpu.async_copy(rows[b], out_hbm.at[pl.ds((row0 + j) * K, K)], so[b])

    def wait_gather(j, b):
        pltpu.make_async_copy(table2_sh.at[cidx_v.at[j]], rows[b], sg[b]).wait()

    def wait_write(j, b):
        pltpu.make_async_copy(
            rows[b], out_hbm.at[pl.ds((row0 + j) * K, K)], so[b]
        ).wait()

    for b in range(NBUF):
        fire_gather(b, b)

    def outer(gi, carry):
        g = gi * NBUF
        for b in range(NBUF):
            j = g + b
            wait_gather(j, b)
            fire_write(j, b)
            wait_write(j, b)

            @pl.when(j + NBUF < NKC)
            def _():
                fire_gather(j + NBUF, b)

        return carry

    lax.fori_loop(0, NKC // NBUF, outer, 0)


_gather_call = functools.partial(
    pl.kernel,
    out_type=jax.ShapeDtypeStruct((TOP * H, 2 * D), jnp.float32),
    mesh=plsc.VectorSubcoreMesh(
        core_axis_name="c", subcore_axis_name="s", num_cores=NC, num_subcores=NS
    ),
    scratch_types=(
        [
            pltpu.VMEM((NROW, N_SIDE), jnp.int32),
            pltpu.VMEM((NKC, K), jnp.int32),
            pltpu.VMEM_SHARED((V * V, 2 * D), jnp.float32),
        ]
        + [pltpu.VMEM((K, 2 * D), jnp.float32)] * NBUF
        + [pltpu.SemaphoreType.DMA] * (2 * NBUF)
    ),
)(_sc_body)


BI = 4


def _tc_body(x_ref, d_ref, tt_ref, o_ref):
    i = pl.program_id(0)

    for r in range(BI):

        @pl.when(i * BI + r < TOP)
        def _():
            for c in range(H // 128):
                t = x_ref[pl.ds(r * H + c * 128, 128), :].T
                o_ref[r, :, pl.ds(c * 128, 128)] = t[:D, :]
                o_ref[r, :, pl.ds(H + c * 128, 128)] = t[D:, :]

        @pl.when(i * BI + r >= TOP)
        def _():
            idx = d_ref[r, 0, :]
            idx = jnp.minimum(jnp.maximum(idx, 0), MAXD)
            onehot = (
                lax.broadcasted_iota(jnp.int32, (VP, N_SIDE), 0) == idx[None, :]
            ).astype(jnp.float32)
            o_ref[r] = lax.dot_general(
                tt_ref[...],
                onehot,
                (((1,), (0,)), ((), ())),
                precision=lax.Precision.HIGHEST,
                preferred_element_type=jnp.float32,
            )


_relayout_call = pl.pallas_call(
    _tc_body,
    grid=(N_SIDE // BI,),
    in_specs=[
        pl.BlockSpec((BI * H, 2 * D), lambda i: (jnp.minimum(i, TOP // BI - 1), 0)),
        pl.BlockSpec((BI, 1, N_SIDE), lambda i: (i, 0, 0)),
        pl.BlockSpec((D, VP), lambda i: (0, 0)),
    ],
    out_specs=pl.BlockSpec((BI, D, N_SIDE), lambda i: (i, 0, 0)),
    out_shape=jax.ShapeDtypeStruct((N_SIDE, D, N_SIDE), jnp.float32),
)


def kernel(distance_matrix, table):
    table2 = jnp.concatenate(
        [jnp.repeat(table, V, axis=0), jnp.tile(table, (V, 1))], axis=1
    )
    paired = _gather_call(distance_matrix, table2)
    tableT = jnp.pad(table, ((0, VP - V), (0, 0))).T
    dist3 = distance_matrix.reshape(N_SIDE, 1, N_SIDE)
    out_t = _relayout_call(paired, dist3, tableT)
    return jnp.transpose(out_t, (0, 2, 1))


# speedup vs baseline: 2.1957x; 1.0596x over previous
import functools

import jax
import jax.numpy as jnp
from jax import lax
from jax.experimental import pallas as pl
from jax.experimental.pallas import tpu as pltpu
from jax.experimental.pallas import tpu_sc as plsc

MAXD = 10
V = MAXD + 1
D = 64
N_SIDE = 1024
H = N_SIDE // 2
TOP = N_SIDE // 4
NC = 2
NS = 16
NW = NC * NS
K = 128
NROW = TOP // NW
NKC = TOP * H // (NW * K)
L = 16
NBUF = 4
VP = 16


def _sc_body(dist_hbm, table2_hbm, out_hbm, idx_v, cidx_v, table2_sh, *bufs_and_sems):
    rows = bufs_and_sems[:NBUF]
    sg = bufs_and_sems[NBUF : 2 * NBUF]
    so = bufs_and_sems[2 * NBUF : 3 * NBUF]

    sid = lax.axis_index("s")
    wid = sid * NC + lax.axis_index("c")
    row0 = wid * NKC

    @pl.when(sid == 0)
    def _():
        pltpu.sync_copy(table2_hbm, table2_sh)

    pltpu.sync_copy(dist_hbm.at[pl.ds(wid * NROW, NROW)], idx_v)

    def combine_row(il, carry):
        for g in range(H // L):
            a = idx_v[il, pl.ds(g * L, L)]
            b = idx_v[il, pl.ds(H + g * L, L)]
            a = jnp.minimum(jnp.maximum(a, 0), MAXD)
            b = jnp.minimum(jnp.maximum(b, 0), MAXD)
            cidx_v[il * (H // K) + g // (K // L), pl.ds((g % (K // L)) * L, L)] = (
                a * V + b
            )
        return carry

    lax.fori_loop(0, NROW, combine_row, 0)

    plsc.subcore_barrier()

    def fire_gather(j, b):
        pltpu.async_copy(table2_sh.at[cidx_v.at[j]], rows[b], sg[b])

    def fire_write(j, b):
        pltpu.async_copy(rows[b], out_hbm.at[pl.ds((row0 + j) * K, K)], so[b])

    def wait_gather(j, b):
        pltpu.make_async_copy(table2_sh.at[cidx_v.at[j]], rows[b], sg[b]).wait()

    def wait_write(j, b):
        pltpu.make_async_copy(
            rows[b], out_hbm.at[pl.ds((row0 + j) * K, K)], so[b]
        ).wait()

    for b in range(NBUF):
        fire_gather(b, b)

    def outer(gi, carry):
        g = gi * NBUF
        for b in range(NBUF):
            j = g + b
            wait_gather(j, b)
            fire_write(j, b)
            wait_write(j, b)

            @pl.when(j + NBUF < NKC)
            def _():
                fire_gather(j + NBUF, b)

        return carry

    lax.fori_loop(0, NKC // NBUF, outer, 0)


_gather_call = functools.partial(
    pl.kernel,
    out_type=jax.ShapeDtypeStruct((TOP * H, 2 * D), jnp.float32),
    mesh=plsc.VectorSubcoreMesh(
        core_axis_name="c", subcore_axis_name="s", num_cores=NC, num_subcores=NS
    ),
    scratch_types=(
        [
            pltpu.VMEM((NROW, N_SIDE), jnp.int32),
            pltpu.VMEM((NKC, K), jnp.int32),
            pltpu.VMEM_SHARED((V * V, 2 * D), jnp.float32),
        ]
        + [pltpu.VMEM((K, 2 * D), jnp.float32)] * NBUF
        + [pltpu.SemaphoreType.DMA] * (2 * NBUF)
    ),
)(_sc_body)


BI = 8


def _tc_body(x_ref, d_ref, tt_ref, o_ref):
    i = pl.program_id(0)

    for r in range(BI):

        @pl.when(i * BI + r < TOP)
        def _():
            for c in range(H // 128):
                t = x_ref[pl.ds(r * H + c * 128, 128), :].T
                o_ref[r, :, pl.ds(c * 128, 128)] = t[:D, :]
                o_ref[r, :, pl.ds(H + c * 128, 128)] = t[D:, :]

        @pl.when(i * BI + r >= TOP)
        def _():
            idx = d_ref[r, 0, :]
            idx = jnp.minimum(jnp.maximum(idx, 0), MAXD)
            onehot = (
                lax.broadcasted_iota(jnp.int32, (VP, N_SIDE), 0) == idx[None, :]
            ).astype(jnp.float32)
            o_ref[r] = lax.dot_general(
                tt_ref[...],
                onehot,
                (((1,), (0,)), ((), ())),
                precision=lax.Precision.HIGHEST,
                preferred_element_type=jnp.float32,
            )


_relayout_call = pl.pallas_call(
    _tc_body,
    grid=(N_SIDE // BI,),
    in_specs=[
        pl.BlockSpec((BI * H, 2 * D), lambda i: (jnp.minimum(i, TOP // BI - 1), 0)),
        pl.BlockSpec((BI, 1, N_SIDE), lambda i: (i, 0, 0)),
        pl.BlockSpec((D, VP), lambda i: (0, 0)),
    ],
    out_specs=pl.BlockSpec((BI, D, N_SIDE), lambda i: (i, 0, 0)),
    out_shape=jax.ShapeDtypeStruct((N_SIDE, D, N_SIDE), jnp.float32),
)


def kernel(distance_matrix, table):
    table2 = jnp.concatenate(
        [jnp.repeat(table, V, axis=0), jnp.tile(table, (V, 1))], axis=1
    )
    paired = _gather_call(distance_matrix, table2)
    tableT = jnp.pad(table, ((0, VP - V), (0, 0))).T
    dist3 = distance_matrix.reshape(N_SIDE, 1, N_SIDE)
    out_t = _relayout_call(paired, dist3, tableT)
    return jnp.transpose(out_t, (0, 2, 1))
